# baseline (speedup 1.0000x reference)
import functools

import jax
import jax.numpy as jnp
from jax import lax
from jax.experimental import pallas as pl
from jax.experimental.pallas import tpu as pltpu
from jax.experimental.pallas import tpu_sc as plsc

VOCAB = 1000000
EMBED = 32
OUT = 64
N = 16384 * 50
NC, NS = 2, 16
NW = NC * NS
B_PER_W = N // NW
CHUNK = 1280
NCHUNK = B_PER_W // CHUNK

BLK = 4096


@functools.lru_cache(maxsize=None)
def _make_gather(n_tokens):
    per_w = n_tokens // NW
    n_chunk = per_w // CHUNK
    assert per_w % CHUNK == 0
    mesh = plsc.VectorSubcoreMesh(
        core_axis_name="c", subcore_axis_name="s", num_cores=NC, num_subcores=NS
    )

    @functools.partial(
        pl.kernel,
        out_type=jax.ShapeDtypeStruct((n_tokens, EMBED), jnp.float32),
        mesh=mesh,
        scratch_types=[
            pltpu.VMEM((CHUNK,), jnp.int32),
            pltpu.VMEM((CHUNK, EMBED), jnp.float32),
            pltpu.SemaphoreType.DMA,
        ],
        compiler_params=pltpu.CompilerParams(use_tc_tiling_on_sc=False),
    )
    def gather(idx_hbm, table_hbm, out_hbm, idx_v, rows_v, sem):
        wid = lax.axis_index("s") * NC + lax.axis_index("c")
        base = wid * per_w

        def step(i, carry):
            off = base + i * CHUNK
            pltpu.sync_copy(idx_hbm.at[pl.ds(off, CHUNK)], idx_v)
            pltpu.async_copy(table_hbm.at[idx_v], rows_v, sem).wait()
            pltpu.sync_copy(rows_v, out_hbm.at[pl.ds(off, CHUNK)])
            return carry

        lax.fori_loop(0, n_chunk, step, 0)

    return gather


PACK = 128 // EMBED
NR = N // PACK
LANES_IN = PACK * EMBED
LANES_OUT = PACK * OUT


def _ln_body(x_ref, wcat_ref, bcat_ref, a_ref, g_ref, bt_ref, out_ref):
    x = x_ref[...]
    hh = (
        jnp.dot(x, wcat_ref[...], preferred_element_type=jnp.float32)
        + bcat_ref[...]
    )
    h = hh[:, :LANES_OUT]
    mu = hh[:, LANES_OUT:]
    d = h - mu
    var = jnp.dot(d * d, a_ref[...], preferred_element_type=jnp.float32)
    out_ref[...] = d * lax.rsqrt(var + 1e-5) * g_ref[...] + bt_ref[...]


def _linear_layernorm(emb128, W, b, gamma, beta):
    nr = emb128.shape[0]
    eye = jnp.eye(PACK, dtype=jnp.float32)
    wbig = jnp.kron(eye, W)
    avg = jnp.kron(eye, jnp.full((OUT, OUT), 1.0 / OUT, jnp.float32))
    wcat = jnp.concatenate([wbig, wbig @ avg], axis=1)
    b4 = jnp.tile(b, PACK).reshape(1, LANES_OUT)
    bcat = jnp.concatenate([b4, b4 @ avg], axis=1)
    g4 = jnp.tile(gamma, PACK).reshape(1, LANES_OUT)
    bt4 = jnp.tile(beta, PACK).reshape(1, LANES_OUT)
    return pl.pallas_call(
        _ln_body,
        grid=(nr // BLK,),
        in_specs=[
            pl.BlockSpec((BLK, LANES_IN), lambda i: (i, 0)),
            pl.BlockSpec((LANES_IN, 2 * LANES_OUT), lambda i: (0, 0)),
            pl.BlockSpec((1, 2 * LANES_OUT), lambda i: (0, 0)),
            pl.BlockSpec((LANES_OUT, LANES_OUT), lambda i: (0, 0)),
            pl.BlockSpec((1, LANES_OUT), lambda i: (0, 0)),
            pl.BlockSpec((1, LANES_OUT), lambda i: (0, 0)),
        ],
        out_specs=pl.BlockSpec((BLK, LANES_OUT), lambda i: (i, 0)),
        out_shape=jax.ShapeDtypeStruct((nr, LANES_OUT), jnp.float32),
    )(emb128, wcat, bcat, avg, g4, bt4)


TSPLIT = 2


@jax.jit
def kernel(texts, table, W, b, gamma, beta):
    Bsz, T = texts.shape
    ts = T // TSPLIT
    outs = []
    for s in range(TSPLIT):
        ids = texts[:, s * ts:(s + 1) * ts].reshape(-1).astype(jnp.int32)
        emb = _make_gather(Bsz * ts)(ids, table)
        emb128 = emb.reshape(Bsz * ts // PACK, LANES_IN)
        o = _linear_layernorm(emb128, W, b, gamma, beta)
        outs.append(o.reshape(Bsz, ts, OUT))
    return jnp.concatenate(outs, axis=1)


# speedup vs baseline: 1.1133x; 1.1133x over previous
import functools

import jax
import jax.numpy as jnp
from jax import lax
from jax.experimental import pallas as pl
from jax.experimental.pallas import tpu as pltpu
from jax.experimental.pallas import tpu_sc as plsc

VOCAB = 1000000
EMBED = 32
OUT = 64
N = 16384 * 50
NC, NS = 2, 16
NW = NC * NS
B_PER_W = N // NW
CHUNK = 1280
NCHUNK = B_PER_W // CHUNK

BLK = 4096


@functools.lru_cache(maxsize=None)
def _make_gather(n_tokens):
    per_w = n_tokens // NW
    n_chunk = per_w // CHUNK
    assert per_w % CHUNK == 0
    mesh = plsc.VectorSubcoreMesh(
        core_axis_name="c", subcore_axis_name="s", num_cores=NC, num_subcores=NS
    )

    @functools.partial(
        pl.kernel,
        out_type=jax.ShapeDtypeStruct((n_tokens, EMBED), jnp.float32),
        mesh=mesh,
        scratch_types=[
            pltpu.VMEM((CHUNK,), jnp.int32),
            pltpu.VMEM((CHUNK, EMBED), jnp.float32),
            pltpu.SemaphoreType.DMA,
        ],
        compiler_params=pltpu.CompilerParams(use_tc_tiling_on_sc=False),
    )
    def gather(idx_hbm, table_hbm, out_hbm, idx_v, rows_v, sem):
        wid = lax.axis_index("s") * NC + lax.axis_index("c")
        base = wid * per_w

        def step(i, carry):
            off = base + i * CHUNK
            pltpu.sync_copy(idx_hbm.at[pl.ds(off, CHUNK)], idx_v)
            pltpu.async_copy(table_hbm.at[idx_v], rows_v, sem).wait()
            pltpu.sync_copy(rows_v, out_hbm.at[pl.ds(off, CHUNK)])
            return carry

        lax.fori_loop(0, n_chunk, step, 0)

    return gather


PACK = 128 // EMBED
NR = N // PACK
LANES_IN = PACK * EMBED
LANES_OUT = PACK * OUT


def _ln_body(x_ref, wcat_ref, bcat_ref, a_ref, g_ref, bt_ref, out_ref):
    x = x_ref[...]
    hh = (
        jnp.dot(x, wcat_ref[...], preferred_element_type=jnp.float32)
        + bcat_ref[...]
    )
    h = hh[:, :LANES_OUT]
    mu = hh[:, LANES_OUT:]
    d = h - mu
    var = jnp.dot(d * d, a_ref[...], preferred_element_type=jnp.float32)
    out_ref[...] = d * lax.rsqrt(var + 1e-5) * g_ref[...] + bt_ref[...]


def _linear_layernorm(emb128, W, b, gamma, beta):
    nr = emb128.shape[0]
    eye = jnp.eye(PACK, dtype=jnp.float32)
    wbig = jnp.kron(eye, W)
    avg = jnp.kron(eye, jnp.full((OUT, OUT), 1.0 / OUT, jnp.float32))
    wcat = jnp.concatenate([wbig, wbig @ avg], axis=1)
    b4 = jnp.tile(b, PACK).reshape(1, LANES_OUT)
    bcat = jnp.concatenate([b4, b4 @ avg], axis=1)
    g4 = jnp.tile(gamma, PACK).reshape(1, LANES_OUT)
    bt4 = jnp.tile(beta, PACK).reshape(1, LANES_OUT)
    return pl.pallas_call(
        _ln_body,
        grid=(nr // BLK,),
        in_specs=[
            pl.BlockSpec((BLK, LANES_IN), lambda i: (i, 0)),
            pl.BlockSpec((LANES_IN, 2 * LANES_OUT), lambda i: (0, 0)),
            pl.BlockSpec((1, 2 * LANES_OUT), lambda i: (0, 0)),
            pl.BlockSpec((LANES_OUT, LANES_OUT), lambda i: (0, 0)),
            pl.BlockSpec((1, LANES_OUT), lambda i: (0, 0)),
            pl.BlockSpec((1, LANES_OUT), lambda i: (0, 0)),
        ],
        out_specs=pl.BlockSpec((BLK, LANES_OUT), lambda i: (i, 0)),
        out_shape=jax.ShapeDtypeStruct((nr, LANES_OUT), jnp.float32),
    )(emb128, wcat, bcat, avg, g4, bt4)


TSPLIT = 1


@jax.jit
def kernel(texts, table, W, b, gamma, beta):
    Bsz, T = texts.shape
    ts = T // TSPLIT
    outs = []
    for s in range(TSPLIT):
        ids = texts[:, s * ts:(s + 1) * ts].reshape(-1).astype(jnp.int32)
        emb = _make_gather(Bsz * ts)(ids, table)
        emb128 = emb.reshape(Bsz * ts // PACK, LANES_IN)
        o = _linear_layernorm(emb128, W, b, gamma, beta)
        outs.append(o.reshape(Bsz, ts, OUT))
    return jnp.concatenate(outs, axis=1)


# speedup vs baseline: 1.4174x; 1.2732x over previous
import functools

import jax
import jax.numpy as jnp
from jax import lax
from jax.experimental import pallas as pl
from jax.experimental.pallas import tpu as pltpu
from jax.experimental.pallas import tpu_sc as plsc

VOCAB = 1000000
EMBED = 32
OUT = 64
N = 16384 * 50
NC, NS = 2, 16
NW = NC * NS
B_PER_W = N // NW
CHUNK = 1280
NCHUNK = B_PER_W // CHUNK

BLK = 4096


@functools.lru_cache(maxsize=None)
def _make_gather(n_tokens):
    per_w = n_tokens // NW
    n_chunk = per_w // CHUNK
    assert per_w % CHUNK == 0
    mesh = plsc.VectorSubcoreMesh(
        core_axis_name="c", subcore_axis_name="s", num_cores=NC, num_subcores=NS
    )

    @functools.partial(
        pl.kernel,
        out_type=jax.ShapeDtypeStruct((n_tokens, EMBED), jnp.float32),
        mesh=mesh,
        scratch_types=[
            pltpu.VMEM((CHUNK,), jnp.int32),
            pltpu.VMEM((CHUNK, EMBED), jnp.float32),
            pltpu.SemaphoreType.DMA,
        ],
        compiler_params=pltpu.CompilerParams(use_tc_tiling_on_sc=False),
    )
    def gather(idx_hbm, table_hbm, out_hbm, idx_v, rows_v, sem):
        wid = lax.axis_index("s") * NC + lax.axis_index("c")
        base = wid * per_w

        def step(i, carry):
            off = base + i * CHUNK
            pltpu.sync_copy(idx_hbm.at[pl.ds(off, CHUNK)], idx_v)
            pltpu.async_copy(table_hbm.at[idx_v], rows_v, sem).wait()
            pltpu.sync_copy(rows_v, out_hbm.at[pl.ds(off, CHUNK)])
            return carry

        lax.fori_loop(0, n_chunk, step, 0)

    return gather


PACK = 128 // EMBED
NR = N // PACK
LANES_IN = PACK * EMBED
LANES_OUT = PACK * OUT


def _ln_body(x_ref, wcat_ref, bcat_ref, a_ref, g_ref, bt_ref, out_ref):
    x = x_ref[...]
    hh = (
        jnp.dot(x, wcat_ref[...], preferred_element_type=jnp.float32)
        + bcat_ref[...]
    )
    h = hh[:, :LANES_OUT]
    mu = hh[:, LANES_OUT:]
    d = h - mu
    var = jnp.dot(d * d, a_ref[...], preferred_element_type=jnp.float32)
    out_ref[...] = d * lax.rsqrt(var + 1e-5) * g_ref[...] + bt_ref[...]


BN = 2048


def _linear_layernorm_t(emb128, W, b, gamma, beta, Bsz, T):
    wmu = W @ jnp.full((OUT, 1), 1.0 / OUT, jnp.float32)
    wct = jnp.concatenate([W.T, wmu.T], axis=0)
    bc = (b - jnp.mean(b)).reshape(OUT, 1)
    ones64 = jnp.full((1, OUT), 1.0 / OUT, jnp.float32)
    gcol = gamma.reshape(OUT, 1)
    btcol = beta.reshape(OUT, 1)
    sub = BN // PACK

    def body(x_ref, wct_ref, bc_ref, ones_ref, g_ref, bt_ref, out_ref):
        x4 = x_ref[...]
        xt = x4.T
        e = jnp.concatenate(
            [xt[j * EMBED:(j + 1) * EMBED] for j in range(PACK)], axis=1
        )
        h65 = jnp.dot(wct_ref[...], e, preferred_element_type=jnp.float32)
        d = h65[:OUT] - h65[OUT:OUT + 1] + bc_ref[...]
        var = jnp.dot(ones_ref[...], d * d, preferred_element_type=jnp.float32)
        y = d * lax.rsqrt(var + 1e-5) * g_ref[...] + bt_ref[...]
        out_ref[0] = y

    nb = Bsz // BN
    return pl.pallas_call(
        body,
        grid=(T, nb),
        in_specs=[
            pl.BlockSpec((sub, LANES_IN), lambda t, j: (t * (Bsz // BN) + j, 0)),
            pl.BlockSpec((OUT + 1, EMBED), lambda t, j: (0, 0)),
            pl.BlockSpec((OUT, 1), lambda t, j: (0, 0)),
            pl.BlockSpec((1, OUT), lambda t, j: (0, 0)),
            pl.BlockSpec((OUT, 1), lambda t, j: (0, 0)),
            pl.BlockSpec((OUT, 1), lambda t, j: (0, 0)),
        ],
        out_specs=pl.BlockSpec((1, OUT, BN), lambda t, j: (t, 0, j)),
        out_shape=jax.ShapeDtypeStruct((T, OUT, Bsz), jnp.float32),
    )(emb128, wct, bc, ones64, gcol, btcol)


def _linear_layernorm(emb128, W, b, gamma, beta):
    nr = emb128.shape[0]
    eye = jnp.eye(PACK, dtype=jnp.float32)
    wbig = jnp.kron(eye, W)
    avg = jnp.kron(eye, jnp.full((OUT, OUT), 1.0 / OUT, jnp.float32))
    wcat = jnp.concatenate([wbig, wbig @ avg], axis=1)
    b4 = jnp.tile(b, PACK).reshape(1, LANES_OUT)
    bcat = jnp.concatenate([b4, b4 @ avg], axis=1)
    g4 = jnp.tile(gamma, PACK).reshape(1, LANES_OUT)
    bt4 = jnp.tile(beta, PACK).reshape(1, LANES_OUT)
    return pl.pallas_call(
        _ln_body,
        grid=(nr // BLK,),
        in_specs=[
            pl.BlockSpec((BLK, LANES_IN), lambda i: (i, 0)),
            pl.BlockSpec((LANES_IN, 2 * LANES_OUT), lambda i: (0, 0)),
            pl.BlockSpec((1, 2 * LANES_OUT), lambda i: (0, 0)),
            pl.BlockSpec((LANES_OUT, LANES_OUT), lambda i: (0, 0)),
            pl.BlockSpec((1, LANES_OUT), lambda i: (0, 0)),
            pl.BlockSpec((1, LANES_OUT), lambda i: (0, 0)),
        ],
        out_specs=pl.BlockSpec((BLK, LANES_OUT), lambda i: (i, 0)),
        out_shape=jax.ShapeDtypeStruct((nr, LANES_OUT), jnp.float32),
    )(emb128, wcat, bcat, avg, g4, bt4)


@jax.jit
def kernel(texts, table, W, b, gamma, beta):
    Bsz, T = texts.shape
    sub = BN // PACK
    ids = jnp.transpose(
        texts.T.reshape(T, Bsz // BN, PACK, sub), (0, 1, 3, 2)
    ).reshape(-1).astype(jnp.int32)
    emb = _make_gather(Bsz * T)(ids, table)
    emb128 = emb.reshape(Bsz * T // PACK, LANES_IN)
    o = _linear_layernorm_t(emb128, W, b, gamma, beta, Bsz, T)
    return jnp.transpose(o, (2, 0, 1))


# speedup vs baseline: 1.7968x; 1.2676x over previous
import functools

import jax
import jax.numpy as jnp
from jax import lax
from jax.experimental import pallas as pl
from jax.experimental.pallas import tpu as pltpu
from jax.experimental.pallas import tpu_sc as plsc

VOCAB = 1000000
EMBED = 32
OUT = 64
N = 16384 * 50
NC, NS = 2, 16
NW = NC * NS
B_PER_W = N // NW
CHUNK = 1280
NCHUNK = B_PER_W // CHUNK

BLK = 4096


@functools.lru_cache(maxsize=None)
def _make_gather(n_tokens):
    per_w = n_tokens // NW
    n_chunk = per_w // CHUNK
    assert per_w % CHUNK == 0
    mesh = plsc.VectorSubcoreMesh(
        core_axis_name="c", subcore_axis_name="s", num_cores=NC, num_subcores=NS
    )

    @functools.partial(
        pl.kernel,
        out_type=jax.ShapeDtypeStruct((n_tokens, EMBED), jnp.float32),
        mesh=mesh,
        scratch_types=[
            pltpu.VMEM((CHUNK,), jnp.int32),
            pltpu.VMEM((CHUNK, EMBED), jnp.float32),
            pltpu.SemaphoreType.DMA,
        ],
        compiler_params=pltpu.CompilerParams(use_tc_tiling_on_sc=False),
    )
    def gather(idx_hbm, table_hbm, out_hbm, idx_v, rows_v, sem):
        wid = lax.axis_index("s") * NC + lax.axis_index("c")
        base = wid * per_w

        def step(i, carry):
            off = base + i * CHUNK
            pltpu.sync_copy(idx_hbm.at[pl.ds(off, CHUNK)], idx_v)
            pltpu.async_copy(table_hbm.at[idx_v], rows_v, sem).wait()
            pltpu.sync_copy(rows_v, out_hbm.at[pl.ds(off, CHUNK)])
            return carry

        lax.fori_loop(0, n_chunk, step, 0)

    return gather


PACK = 128 // EMBED
NR = N // PACK
LANES_IN = PACK * EMBED
LANES_OUT = PACK * OUT


def _ln_body(x_ref, wcat_ref, bcat_ref, a_ref, g_ref, bt_ref, out_ref):
    x = x_ref[...]
    hh = (
        jnp.dot(x, wcat_ref[...], preferred_element_type=jnp.float32)
        + bcat_ref[...]
    )
    h = hh[:, :LANES_OUT]
    mu = hh[:, LANES_OUT:]
    d = h - mu
    var = jnp.dot(d * d, a_ref[...], preferred_element_type=jnp.float32)
    out_ref[...] = d * lax.rsqrt(var + 1e-5) * g_ref[...] + bt_ref[...]


BN = 16384


def _linear_layernorm_t(emb128, W, b, gamma, beta, Bsz, T):
    wmu = W @ jnp.full((OUT, 1), 1.0 / OUT, jnp.float32)
    wct = jnp.concatenate([W.T, wmu.T], axis=0)
    bc = (b - jnp.mean(b)).reshape(OUT, 1)
    ones64 = jnp.full((1, OUT), 1.0 / OUT, jnp.float32)
    gcol = gamma.reshape(OUT, 1)
    btcol = beta.reshape(OUT, 1)
    sub = BN // PACK

    def body(x_ref, wct_ref, bc_ref, ones_ref, g_ref, bt_ref, out_ref):
        x4 = x_ref[...]
        xt = x4.T
        e = jnp.concatenate(
            [xt[j * EMBED:(j + 1) * EMBED] for j in range(PACK)], axis=1
        )
        h65 = jnp.dot(wct_ref[...], e, preferred_element_type=jnp.float32)
        d = h65[:OUT] - h65[OUT:OUT + 1] + bc_ref[...]
        var = jnp.dot(ones_ref[...], d * d, preferred_element_type=jnp.float32)
        y = d * lax.rsqrt(var + 1e-5) * g_ref[...] + bt_ref[...]
        out_ref[0] = y

    nb = Bsz // BN
    return pl.pallas_call(
        body,
        grid=(T, nb),
        in_specs=[
            pl.BlockSpec((sub, LANES_IN), lambda t, j: (t * (Bsz // BN) + j, 0)),
            pl.BlockSpec((OUT + 1, EMBED), lambda t, j: (0, 0)),
            pl.BlockSpec((OUT, 1), lambda t, j: (0, 0)),
            pl.BlockSpec((1, OUT), lambda t, j: (0, 0)),
            pl.BlockSpec((OUT, 1), lambda t, j: (0, 0)),
            pl.BlockSpec((OUT, 1), lambda t, j: (0, 0)),
        ],
        out_specs=pl.BlockSpec((1, OUT, BN), lambda t, j: (t, 0, j)),
        out_shape=jax.ShapeDtypeStruct((T, OUT, Bsz), jnp.float32),
    )(emb128, wct, bc, ones64, gcol, btcol)


def _linear_layernorm(emb128, W, b, gamma, beta):
    nr = emb128.shape[0]
    eye = jnp.eye(PACK, dtype=jnp.float32)
    wbig = jnp.kron(eye, W)
    avg = jnp.kron(eye, jnp.full((OUT, OUT), 1.0 / OUT, jnp.float32))
    wcat = jnp.concatenate([wbig, wbig @ avg], axis=1)
    b4 = jnp.tile(b, PACK).reshape(1, LANES_OUT)
    bcat = jnp.concatenate([b4, b4 @ avg], axis=1)
    g4 = jnp.tile(gamma, PACK).reshape(1, LANES_OUT)
    bt4 = jnp.tile(beta, PACK).reshape(1, LANES_OUT)
    return pl.pallas_call(
        _ln_body,
        grid=(nr // BLK,),
        in_specs=[
            pl.BlockSpec((BLK, LANES_IN), lambda i: (i, 0)),
            pl.BlockSpec((LANES_IN, 2 * LANES_OUT), lambda i: (0, 0)),
            pl.BlockSpec((1, 2 * LANES_OUT), lambda i: (0, 0)),
            pl.BlockSpec((LANES_OUT, LANES_OUT), lambda i: (0, 0)),
            pl.BlockSpec((1, LANES_OUT), lambda i: (0, 0)),
            pl.BlockSpec((1, LANES_OUT), lambda i: (0, 0)),
        ],
        out_specs=pl.BlockSpec((BLK, LANES_OUT), lambda i: (i, 0)),
        out_shape=jax.ShapeDtypeStruct((nr, LANES_OUT), jnp.float32),
    )(emb128, wcat, bcat, avg, g4, bt4)


@jax.jit
def kernel(texts, table, W, b, gamma, beta):
    Bsz, T = texts.shape
    sub = BN // PACK
    ids = jnp.transpose(
        texts.T.reshape(T, Bsz // BN, PACK, sub), (0, 1, 3, 2)
    ).reshape(-1).astype(jnp.int32)
    emb = _make_gather(Bsz * T)(ids, table)
    emb128 = emb.reshape(Bsz * T // PACK, LANES_IN)
    o = _linear_layernorm_t(emb128, W, b, gamma, beta, Bsz, T)
    return jnp.transpose(o, (2, 0, 1))


# speedup vs baseline: 1.8311x; 1.0191x over previous
import functools

import jax
import jax.numpy as jnp
from jax import lax
from jax.experimental import pallas as pl
from jax.experimental.pallas import tpu as pltpu
from jax.experimental.pallas import tpu_sc as plsc

VOCAB = 1000000
EMBED = 32
OUT = 64
N = 16384 * 50
NC, NS = 2, 16
NW = NC * NS
B_PER_W = N // NW
CHUNK = 1280
NCHUNK = B_PER_W // CHUNK

BLK = 4096


@functools.lru_cache(maxsize=None)
def _make_gather(n_tokens):
    per_w = n_tokens // NW
    n_chunk = per_w // CHUNK
    assert per_w % CHUNK == 0
    mesh = plsc.VectorSubcoreMesh(
        core_axis_name="c", subcore_axis_name="s", num_cores=NC, num_subcores=NS
    )

    @functools.partial(
        pl.kernel,
        out_type=jax.ShapeDtypeStruct((n_tokens, EMBED), jnp.float32),
        mesh=mesh,
        scratch_types=[
            pltpu.VMEM((2, CHUNK), jnp.int32),
            pltpu.VMEM((2 * CHUNK, EMBED), jnp.float32),
            pltpu.SemaphoreType.DMA,
            pltpu.SemaphoreType.DMA,
        ],
        compiler_params=pltpu.CompilerParams(use_tc_tiling_on_sc=False),
    )
    def gather(idx_hbm, table_hbm, out_hbm, idx_v, rows_v, sem0, sem1):
        wid = lax.axis_index("s") * NC + lax.axis_index("c")
        base = wid * per_w
        sems = (sem0, sem1)

        def start(i):
            p = i % 2
            pltpu.sync_copy(
                idx_hbm.at[pl.ds(base + i * CHUNK, CHUNK)], idx_v.at[p]
            )
            return pltpu.async_copy(
                table_hbm.at[idx_v.at[p]],
                rows_v.at[pl.ds(p * CHUNK, CHUNK)],
                sems[p],
            )

        cps = [start(0), start(1)]
        for i in range(n_chunk):
            p = i % 2
            cps[p].wait()
            pltpu.sync_copy(
                rows_v.at[pl.ds(p * CHUNK, CHUNK)],
                out_hbm.at[pl.ds(base + i * CHUNK, CHUNK)],
            )
            if i + 2 < n_chunk:
                cps[p] = start(i + 2)

    return gather


PACK = 128 // EMBED
NR = N // PACK
LANES_IN = PACK * EMBED
LANES_OUT = PACK * OUT


def _ln_body(x_ref, wcat_ref, bcat_ref, a_ref, g_ref, bt_ref, out_ref):
    x = x_ref[...]
    hh = (
        jnp.dot(x, wcat_ref[...], preferred_element_type=jnp.float32)
        + bcat_ref[...]
    )
    h = hh[:, :LANES_OUT]
    mu = hh[:, LANES_OUT:]
    d = h - mu
    var = jnp.dot(d * d, a_ref[...], preferred_element_type=jnp.float32)
    out_ref[...] = d * lax.rsqrt(var + 1e-5) * g_ref[...] + bt_ref[...]


BN = 16384


def _linear_layernorm_t(emb128, W, b, gamma, beta, Bsz, T):
    wmu = W @ jnp.full((OUT, 1), 1.0 / OUT, jnp.float32)
    wct = jnp.concatenate([W.T, wmu.T], axis=0)
    bc = (b - jnp.mean(b)).reshape(OUT, 1)
    ones64 = jnp.full((1, OUT), 1.0 / OUT, jnp.float32)
    gcol = gamma.reshape(OUT, 1)
    btcol = beta.reshape(OUT, 1)
    sub = BN // PACK

    def body(x_ref, wct_ref, bc_ref, ones_ref, g_ref, bt_ref, out_ref):
        x4 = x_ref[...]
        xt = x4.T
        e = jnp.concatenate(
            [xt[j * EMBED:(j + 1) * EMBED] for j in range(PACK)], axis=1
        )
        h65 = jnp.dot(wct_ref[...], e, preferred_element_type=jnp.float32)
        d = h65[:OUT] - h65[OUT:OUT + 1] + bc_ref[...]
        var = jnp.dot(ones_ref[...], d * d, preferred_element_type=jnp.float32)
        y = d * lax.rsqrt(var + 1e-5) * g_ref[...] + bt_ref[...]
        out_ref[0] = y

    nb = Bsz // BN
    return pl.pallas_call(
        body,
        grid=(T, nb),
        in_specs=[
            pl.BlockSpec((sub, LANES_IN), lambda t, j: (t * (Bsz // BN) + j, 0)),
            pl.BlockSpec((OUT + 1, EMBED), lambda t, j: (0, 0)),
            pl.BlockSpec((OUT, 1), lambda t, j: (0, 0)),
            pl.BlockSpec((1, OUT), lambda t, j: (0, 0)),
            pl.BlockSpec((OUT, 1), lambda t, j: (0, 0)),
            pl.BlockSpec((OUT, 1), lambda t, j: (0, 0)),
        ],
        out_specs=pl.BlockSpec((1, OUT, BN), lambda t, j: (t, 0, j)),
        out_shape=jax.ShapeDtypeStruct((T, OUT, Bsz), jnp.float32),
    )(emb128, wct, bc, ones64, gcol, btcol)


def _linear_layernorm(emb128, W, b, gamma, beta):
    nr = emb128.shape[0]
    eye = jnp.eye(PACK, dtype=jnp.float32)
    wbig = jnp.kron(eye, W)
    avg = jnp.kron(eye, jnp.full((OUT, OUT), 1.0 / OUT, jnp.float32))
    wcat = jnp.concatenate([wbig, wbig @ avg], axis=1)
    b4 = jnp.tile(b, PACK).reshape(1, LANES_OUT)
    bcat = jnp.concatenate([b4, b4 @ avg], axis=1)
    g4 = jnp.tile(gamma, PACK).reshape(1, LANES_OUT)
    bt4 = jnp.tile(beta, PACK).reshape(1, LANES_OUT)
    return pl.pallas_call(
        _ln_body,
        grid=(nr // BLK,),
        in_specs=[
            pl.BlockSpec((BLK, LANES_IN), lambda i: (i, 0)),
            pl.BlockSpec((LANES_IN, 2 * LANES_OUT), lambda i: (0, 0)),
            pl.BlockSpec((1, 2 * LANES_OUT), lambda i: (0, 0)),
            pl.BlockSpec((LANES_OUT, LANES_OUT), lambda i: (0, 0)),
            pl.BlockSpec((1, LANES_OUT), lambda i: (0, 0)),
            pl.BlockSpec((1, LANES_OUT), lambda i: (0, 0)),
        ],
        out_specs=pl.BlockSpec((BLK, LANES_OUT), lambda i: (i, 0)),
        out_shape=jax.ShapeDtypeStruct((nr, LANES_OUT), jnp.float32),
    )(emb128, wcat, bcat, avg, g4, bt4)


@jax.jit
def kernel(texts, table, W, b, gamma, beta):
    Bsz, T = texts.shape
    sub = BN // PACK
    ids = jnp.transpose(
        texts.T.reshape(T, Bsz // BN, PACK, sub), (0, 1, 3, 2)
    ).reshape(-1).astype(jnp.int32)
    emb = _make_gather(Bsz * T)(ids, table)
    emb128 = emb.reshape(Bsz * T // PACK, LANES_IN)
    o = _linear_layernorm_t(emb128, W, b, gamma, beta, Bsz, T)
    return jnp.transpose(o, (2, 0, 1))
